# speedup vs baseline: 1.7129x; 1.7129x over previous
import functools

import jax
import jax.numpy as jnp
from jax import lax
from jax.experimental import pallas as pl
from jax.experimental.pallas import tpu as pltpu
from jax.experimental.pallas import tpu_sc as plsc

_INFO = plsc.get_sparse_core_info()
_NC = _INFO.num_cores
_NS = _INFO.num_subcores
_NW = _NC * _NS


@functools.partial(jax.jit, static_argnums=(2, 3))
def _gather_rows(table, idx3, chunk, n_chunks):
    n_rows = _NW * n_chunks * chunk
    d = table.shape[1]
    mesh = plsc.VectorSubcoreMesh(core_axis_name="c", subcore_axis_name="s")

    @functools.partial(
        pl.kernel,
        mesh=mesh,
        out_type=jax.ShapeDtypeStruct((n_rows, d), jnp.float32),
        scratch_types=[
            pltpu.VMEM((n_chunks, chunk), jnp.int32),
            pltpu.VMEM((chunk, d), jnp.float32),
            pltpu.SemaphoreType.DMA,
        ],
    )
    def k(table_hbm, idx_hbm, out_hbm, idx_v, rows_v, sem):
        wid = lax.axis_index("s") * _NC + lax.axis_index("c")
        base = wid * (n_chunks * chunk)
        pltpu.sync_copy(idx_hbm.at[wid], idx_v)

        def body(i, carry):
            pltpu.async_copy(table_hbm.at[idx_v.at[i]], rows_v, sem).wait()
            pltpu.sync_copy(rows_v, out_hbm.at[pl.ds(base + i * chunk, chunk)])
            return carry

        lax.fori_loop(0, n_chunks, body, 0)

    return k(table, idx3)


def kernel(x, emb_weight):
    b, s = x.shape
    n = b * s
    chunk = 16
    n_chunks = n // (_NW * chunk)
    idx3 = x.reshape(_NW, n_chunks, chunk).astype(jnp.int32)
    out = _gather_rows(emb_weight, idx3, chunk, n_chunks)
    return out.reshape(b, s, emb_weight.shape[1])


# speedup vs baseline: 1.7895x; 1.0448x over previous
import functools

import jax
import jax.numpy as jnp
from jax import lax
from jax.experimental import pallas as pl
from jax.experimental.pallas import tpu as pltpu
from jax.experimental.pallas import tpu_sc as plsc

_INFO = plsc.get_sparse_core_info()
_NC = _INFO.num_cores
_NS = _INFO.num_subcores
_NW = _NC * _NS
_NBUF = 2


@functools.partial(jax.jit, static_argnums=(2, 3))
def _gather_rows(table, idx3, chunk, n_chunks):
    n_rows = _NW * n_chunks * chunk
    d = table.shape[1]
    mesh = plsc.VectorSubcoreMesh(core_axis_name="c", subcore_axis_name="s")

    @functools.partial(
        pl.kernel,
        mesh=mesh,
        out_type=jax.ShapeDtypeStruct((n_rows, d), jnp.float32),
        scratch_types=[
            pltpu.VMEM((n_chunks, chunk), jnp.int32),
            pltpu.VMEM((chunk, d), jnp.float32),
            pltpu.VMEM((chunk, d), jnp.float32),
            pltpu.SemaphoreType.DMA,
            pltpu.SemaphoreType.DMA,
            pltpu.SemaphoreType.DMA,
            pltpu.SemaphoreType.DMA,
        ],
    )
    def k(table_hbm, idx_hbm, out_hbm, idx_v, rows0, rows1, g0, g1, o0, o1):
        wid = lax.axis_index("s") * _NC + lax.axis_index("c")
        base = wid * (n_chunks * chunk)
        rows = (rows0, rows1)
        gsem = (g0, g1)
        osem = (o0, o1)
        pltpu.sync_copy(idx_hbm.at[wid], idx_v)

        def start_gather(i, b):
            pltpu.async_copy(table_hbm.at[idx_v.at[i]], rows[b], gsem[b])

        def wait_gather(b):
            pltpu.make_async_copy(
                table_hbm.at[pl.ds(0, chunk)], rows[b], gsem[b]).wait()

        def start_out(i, b):
            pltpu.async_copy(
                rows[b], out_hbm.at[pl.ds(base + i * chunk, chunk)], osem[b])

        def wait_out(i, b):
            pltpu.make_async_copy(
                rows[b], out_hbm.at[pl.ds(base + i * chunk, chunk)],
                osem[b]).wait()

        for b in range(_NBUF):
            start_gather(b, b)

        def body(g, carry):
            i0 = g * _NBUF
            for b in range(_NBUF):
                i = i0 + b
                wait_gather(b)
                start_out(i, b)
                wait_out(i, b)
                start_gather(i + _NBUF, b)
            return carry

        lax.fori_loop(0, n_chunks // _NBUF - 1, body, 0)

        i0 = n_chunks - _NBUF
        for b in range(_NBUF):
            i = i0 + b
            wait_gather(b)
            start_out(i, b)
        for b in range(_NBUF):
            wait_out(i0 + b, b)

    return k(table, idx3)


def kernel(x, emb_weight):
    b, s = x.shape
    n = b * s
    chunk = 8
    n_chunks = n // (_NW * chunk)
    idx3 = x.reshape(_NW, n_chunks, chunk).astype(jnp.int32)
    out = _gather_rows(emb_weight, idx3, chunk, n_chunks)
    return out.reshape(b, s, emb_weight.shape[1])
